# baseline (speedup 1.0000x reference)
import jax
import jax.numpy as jnp
from jax import lax
from jax.experimental import pallas as pl
from jax.experimental.pallas import tpu as pltpu

BANK = 16
ALPHA = 0.1
SIM_HIGH = 0.8
SIM_LOW = 0.3
TEMP = 1.0


def _ge0(x):
    return 1.0 - jnp.maximum(jnp.sign(-x), 0.0)


def _row_update(v, w, p, validf, fb, pg, fg):
    C, HW = v.shape
    K = p.shape[0]
    fK = jnp.float32(K)

    cand = jnp.sum(v * w, axis=1, keepdims=True)
    cn2 = jnp.sum(cand * cand, axis=0, keepdims=True)
    cand = cand / jnp.clip(jnp.sqrt(cn2), 1e-12, None)
    cand_r = cand.reshape(1, C)

    pn2 = jnp.sum(p * p, axis=1, keepdims=True).reshape(1, K)
    pnorm = jnp.clip(jnp.sqrt(pn2), 1e-12, None)
    dots = lax.dot_general(cand_r, p, (((1,), (1,)), ((), ())))
    sim = dots / pnorm
    sim_m = validf * sim - (1.0 - validf) * 1e9
    best_sim = jnp.max(sim_m, axis=1, keepdims=True)
    ki = lax.broadcasted_iota(jnp.int32, (1, K), 1).astype(jnp.float32)
    best_idx = jnp.min(ki + jnp.sign(best_sim - sim_m) * fK,
                       axis=1, keepdims=True)
    any_valid = jnp.minimum(jnp.sum(validf, axis=1, keepdims=True), 1.0)
    spawn = jnp.min(ki + validf * fK, axis=1, keepdims=True)
    spawn = spawn * (1.0 - jnp.maximum(spawn - (fK - 1.0), 0.0))
    refine = any_valid * _ge0(best_sim - SIM_HIGH)
    write = (1.0 - any_valid) + any_valid * _ge0(SIM_LOW - best_sim)
    slot = refine * best_idx + (1.0 - refine) * spawn

    ohrow = jnp.maximum(1.0 - jnp.abs(ki - slot), 0.0)
    kcol = lax.broadcasted_iota(jnp.int32, (K, 1), 0).astype(jnp.float32)
    ohcol = jnp.maximum(1.0 - jnp.abs(kcol - slot), 0.0)
    old_r = jnp.sum(p * ohcol, axis=0, keepdims=True)
    mixed = (1.0 - ALPHA) * old_r + ALPHA * cand_r
    bn2 = jnp.sum(mixed * mixed, axis=1, keepdims=True)
    blended = mixed / jnp.clip(jnp.sqrt(bn2), 1e-12, None)
    newvec = (refine * blended
              + (1.0 - refine) * (write * cand_r + (1.0 - write) * old_r))
    p_new = ohcol * newvec + (1.0 - ohcol) * p
    slot_valid = jnp.sum(validf * ohrow, axis=1, keepdims=True)
    slot_valid_new = jnp.maximum(slot_valid, jnp.maximum(refine, write))
    valid_new = validf + ohrow * (slot_valid_new - slot_valid)

    sim2 = lax.dot_general(cand_r, p_new, (((1,), (1,)), ((), ())))
    logits = valid_new * (sim2 / TEMP) - (1.0 - valid_new) * 1e9
    lmax = jnp.max(logits, axis=1, keepdims=True)
    e = jnp.exp(logits - lmax)
    weights = e / jnp.sum(e, axis=1, keepdims=True)
    pf = lax.dot_general(p_new, weights, (((0,), (1,)), ((), ())))

    return fg * (v + fb) + pg * pf


def _fused_body(v_ref, w_ref, f_ref, p_ref, val_ref, pg_ref, fg_ref, o_ref):
    fb = f_ref[0]
    pg = pg_ref[:, :]
    fg = fg_ref[:, :]
    for j in range(v_ref.shape[1]):
        o_ref[0, j] = _row_update(v_ref[0, j], w_ref[0, j], p_ref[0, j],
                                  val_ref[0, j], fb, pg, fg)


def kernel(value_BNCHW, frame_feat_BCHW, mask_BNHW, proto, valid, proto_gate, frame_gate):
    B, N, C, H, W = value_BNCHW.shape
    K = proto.shape[2]
    HW = H * W
    v = value_BNCHW.reshape(B, N, C, HW)
    f = frame_feat_BCHW.reshape(B, C, HW)
    m = mask_BNHW.reshape(B, N, 1, HW)
    msum = m.sum(axis=3, keepdims=True)
    denom = jnp.clip(msum, 1e-6, None)
    use_fb = denom <= 1e-5
    w = jnp.where(use_fb, jnp.float32(1.0 / HW), m / denom)
    validf = valid.astype(jnp.float32).reshape(B, N, 1, K)
    pg = jnp.reshape(proto_gate, (1, 1)).astype(jnp.float32)
    fg = jnp.reshape(frame_gate, (1, 1)).astype(jnp.float32)

    NT = 4
    grid = (B, N // NT)
    out = pl.pallas_call(
        _fused_body,
        grid=grid,
        in_specs=[
            pl.BlockSpec((1, NT, C, HW), lambda b, n: (b, n, 0, 0)),
            pl.BlockSpec((1, NT, 1, HW), lambda b, n: (b, n, 0, 0)),
            pl.BlockSpec((1, C, HW), lambda b, n: (b, 0, 0)),
            pl.BlockSpec((1, NT, K, C), lambda b, n: (b, n, 0, 0)),
            pl.BlockSpec((1, NT, 1, K), lambda b, n: (b, n, 0, 0)),
            pl.BlockSpec((1, 1), lambda b, n: (0, 0)),
            pl.BlockSpec((1, 1), lambda b, n: (0, 0)),
        ],
        out_specs=pl.BlockSpec((1, NT, C, HW), lambda b, n: (b, n, 0, 0)),
        out_shape=jax.ShapeDtypeStruct((B, N, C, HW), jnp.float32),
    )(v, w, f, proto, validf, pg, fg)
    return out.reshape(B, N, C, H, W)


# speedup vs baseline: 1.0244x; 1.0244x over previous
import jax
import jax.numpy as jnp
from jax import lax
from jax.experimental import pallas as pl
from jax.experimental.pallas import tpu as pltpu

BANK = 16
ALPHA = 0.1
SIM_HIGH = 0.8
SIM_LOW = 0.3
TEMP = 1.0


def _ge0(x):
    return 1.0 - jnp.maximum(jnp.sign(-x), 0.0)


def _row_update(v, w, p, validf, fb, pg, fg):
    C, HW = v.shape
    K = p.shape[0]
    fK = jnp.float32(K)

    cand = jnp.sum(v * w, axis=1, keepdims=True)
    cn2 = jnp.sum(cand * cand, axis=0, keepdims=True)
    cand = cand / jnp.clip(jnp.sqrt(cn2), 1e-12, None)
    cand_r = cand.reshape(1, C)

    pn2 = jnp.sum(p * p, axis=1, keepdims=True).reshape(1, K)
    pnorm = jnp.clip(jnp.sqrt(pn2), 1e-12, None)
    dots = lax.dot_general(cand_r, p, (((1,), (1,)), ((), ())))
    sim = dots / pnorm
    sim_m = validf * sim - (1.0 - validf) * 1e9
    best_sim = jnp.max(sim_m, axis=1, keepdims=True)
    ki = lax.broadcasted_iota(jnp.int32, (1, K), 1).astype(jnp.float32)
    best_idx = jnp.min(ki + jnp.sign(best_sim - sim_m) * fK,
                       axis=1, keepdims=True)
    any_valid = jnp.minimum(jnp.sum(validf, axis=1, keepdims=True), 1.0)
    spawn = jnp.min(ki + validf * fK, axis=1, keepdims=True)
    spawn = spawn * (1.0 - jnp.maximum(spawn - (fK - 1.0), 0.0))
    refine = any_valid * _ge0(best_sim - SIM_HIGH)
    write = (1.0 - any_valid) + any_valid * _ge0(SIM_LOW - best_sim)
    slot = refine * best_idx + (1.0 - refine) * spawn

    ohrow = jnp.maximum(1.0 - jnp.abs(ki - slot), 0.0)
    kcol = lax.broadcasted_iota(jnp.int32, (K, 1), 0).astype(jnp.float32)
    ohcol = jnp.maximum(1.0 - jnp.abs(kcol - slot), 0.0)
    old_r = jnp.sum(p * ohcol, axis=0, keepdims=True)
    mixed = (1.0 - ALPHA) * old_r + ALPHA * cand_r
    bn2 = jnp.sum(mixed * mixed, axis=1, keepdims=True)
    blended = mixed / jnp.clip(jnp.sqrt(bn2), 1e-12, None)
    newvec = (refine * blended
              + (1.0 - refine) * (write * cand_r + (1.0 - write) * old_r))
    p_new = ohcol * newvec + (1.0 - ohcol) * p
    slot_valid = jnp.sum(validf * ohrow, axis=1, keepdims=True)
    slot_valid_new = jnp.maximum(slot_valid, jnp.maximum(refine, write))
    valid_new = validf + ohrow * (slot_valid_new - slot_valid)

    sim2 = lax.dot_general(cand_r, p_new, (((1,), (1,)), ((), ())))
    logits = valid_new * (sim2 / TEMP) - (1.0 - valid_new) * 1e9
    lmax = jnp.max(logits, axis=1, keepdims=True)
    e = jnp.exp(logits - lmax)
    weights = e / jnp.sum(e, axis=1, keepdims=True)
    pf = lax.dot_general(p_new, weights, (((0,), (1,)), ((), ())))

    return fg * (v + fb) + pg * pf


def _fused_body(v_ref, w_ref, f_ref, p_ref, val_ref, pg_ref, fg_ref, o_ref):
    fb = f_ref[0]
    pg = pg_ref[:, :]
    fg = fg_ref[:, :]
    for j in range(v_ref.shape[1]):
        o_ref[0, j] = _row_update(v_ref[0, j], w_ref[0, j], p_ref[0, j],
                                  val_ref[0, j], fb, pg, fg)


def kernel(value_BNCHW, frame_feat_BCHW, mask_BNHW, proto, valid, proto_gate, frame_gate):
    B, N, C, H, W = value_BNCHW.shape
    K = proto.shape[2]
    HW = H * W
    v = value_BNCHW.reshape(B, N, C, HW)
    f = frame_feat_BCHW.reshape(B, C, HW)
    m = mask_BNHW.reshape(B, N, 1, HW)
    msum = m.sum(axis=3, keepdims=True)
    denom = jnp.clip(msum, 1e-6, None)
    use_fb = denom <= 1e-5
    w = jnp.where(use_fb, jnp.float32(1.0 / HW), m / denom)
    validf = valid.astype(jnp.float32).reshape(B, N, 1, K)
    pg = jnp.reshape(proto_gate, (1, 1)).astype(jnp.float32)
    fg = jnp.reshape(frame_gate, (1, 1)).astype(jnp.float32)

    NT = 8
    grid = (B, N // NT)
    out = pl.pallas_call(
        _fused_body,
        grid=grid,
        in_specs=[
            pl.BlockSpec((1, NT, C, HW), lambda b, n: (b, n, 0, 0)),
            pl.BlockSpec((1, NT, 1, HW), lambda b, n: (b, n, 0, 0)),
            pl.BlockSpec((1, C, HW), lambda b, n: (b, 0, 0)),
            pl.BlockSpec((1, NT, K, C), lambda b, n: (b, n, 0, 0)),
            pl.BlockSpec((1, NT, 1, K), lambda b, n: (b, n, 0, 0)),
            pl.BlockSpec((1, 1), lambda b, n: (0, 0)),
            pl.BlockSpec((1, 1), lambda b, n: (0, 0)),
        ],
        out_specs=pl.BlockSpec((1, NT, C, HW), lambda b, n: (b, n, 0, 0)),
        out_shape=jax.ShapeDtypeStruct((B, N, C, HW), jnp.float32),
    )(v, w, f, proto, validf, pg, fg)
    return out.reshape(B, N, C, H, W)
